# baseline (speedup 1.0000x reference)
import functools
import jax
import jax.numpy as jnp
from jax import lax
from jax.experimental import pallas as pl
from jax.experimental.pallas import tpu as pltpu
from jax.experimental.pallas import tpu_sc as plsc


def kernel(x, table):
    B = x.shape[0]
    T, D = table.shape
    info = plsc.get_sparse_core_info()
    NW = info.num_cores * info.num_subcores
    rows_per_w = T // NW
    CH = 32
    nch = rows_per_w // CH
    npair = nch // 2

    mesh = plsc.VectorSubcoreMesh(core_axis_name="c", subcore_axis_name="s")

    @functools.partial(
        pl.kernel,
        mesh=mesh,
        out_type=jax.ShapeDtypeStruct((B, T, D), jnp.float32),
        scratch_types=[
            pltpu.VMEM((CH, D), jnp.float32),
            pltpu.VMEM((CH, D), jnp.float32),
            pltpu.SemaphoreType.DMA,
            pltpu.SemaphoreType.DMA,
            pltpu.SemaphoreType.DMA,
            pltpu.SemaphoreType.DMA,
        ],
    )
    def k(table_hbm, out_hbm, buf0, buf1, rsem0, rsem1, wsem0, wsem1):
        wid = lax.axis_index("s") * info.num_cores + lax.axis_index("c")
        base = wid * rows_per_w

        def rd(r0, buf, rsem):
            return pltpu.make_async_copy(
                table_hbm.at[pl.ds(r0, CH)], buf, rsem
            )

        def wr(r0, buf, wsem, b):
            return pltpu.make_async_copy(
                buf, out_hbm.at[b, pl.ds(r0, CH)], wsem
            )

        rd(base, buf0, rsem0).start()

        def body(j, carry):
            r0 = base + (2 * j) * CH
            r1 = r0 + CH
            rd(r0, buf0, rsem0).wait()

            @pl.when(j >= 1)
            def _():
                for b in range(B):
                    wr(r0 - CH, buf1, wsem1, b).wait()

            rd(r1, buf1, rsem1).start()
            for b in range(B):
                wr(r0, buf0, wsem0, b).start()
            rd(r1, buf1, rsem1).wait()

            @pl.when(j < npair - 1)
            def _():
                for b in range(B):
                    wr(r0, buf0, wsem0, b).wait()
                rd(r1 + CH, buf0, rsem0).start()

            for b in range(B):
                wr(r1, buf1, wsem1, b).start()
            return carry

        lax.fori_loop(0, npair, body, 0)

        last0 = base + (nch - 2) * CH
        for b in range(B):
            wr(last0, buf0, wsem0, b).wait()
        for b in range(B):
            wr(last0 + CH, buf1, wsem1, b).wait()

    return k(table)


# speedup vs baseline: 1.0133x; 1.0133x over previous
import functools
import jax
import jax.numpy as jnp
from jax import lax
from jax.experimental import pallas as pl
from jax.experimental.pallas import tpu as pltpu
from jax.experimental.pallas import tpu_sc as plsc


def kernel(x, table):
    B = x.shape[0]
    T, D = table.shape
    info = plsc.get_sparse_core_info()
    NW = info.num_cores * info.num_subcores
    rows_per_w = T // NW
    CH = 56
    sizes = [56, 56, 56, 56, 32]
    offs = [0, 56, 112, 168, 224]
    nch = len(sizes)

    mesh = plsc.VectorSubcoreMesh(core_axis_name="c", subcore_axis_name="s")

    @functools.partial(
        pl.kernel,
        mesh=mesh,
        out_type=jax.ShapeDtypeStruct((B, T, D), jnp.float32),
        scratch_types=[
            pltpu.VMEM((CH, D), jnp.float32),
            pltpu.VMEM((CH, D), jnp.float32),
            pltpu.SemaphoreType.DMA,
            pltpu.SemaphoreType.DMA,
            pltpu.SemaphoreType.DMA,
            pltpu.SemaphoreType.DMA,
        ],
    )
    def k(table_hbm, out_hbm, buf0, buf1, rsem0, rsem1, wsem0, wsem1):
        wid = lax.axis_index("s") * info.num_cores + lax.axis_index("c")
        base = wid * rows_per_w
        bufs = (buf0, buf1)
        rsems = (rsem0, rsem1)
        wsems = (wsem0, wsem1)

        def rd(c, s):
            return pltpu.make_async_copy(
                table_hbm.at[pl.ds(base + offs[c], sizes[c])],
                bufs[s].at[pl.ds(0, sizes[c])],
                rsems[s],
            )

        def wr(c, s, b):
            return pltpu.make_async_copy(
                bufs[s].at[pl.ds(0, sizes[c])],
                out_hbm.at[b, pl.ds(base + offs[c], sizes[c])],
                wsems[s],
            )

        rd(0, 0).start()

        for c in range(nch):
            s = c % 2
            ns = (c + 1) % 2
            rd(c, s).wait()
            if c + 1 < nch:
                if c >= 1:
                    for b in range(B):
                        wr(c - 1, ns, b).wait()
                rd(c + 1, ns).start()
            for b in range(B):
                wr(c, s, b).start()

        for c in (nch - 2, nch - 1):
            for b in range(B):
                wr(c, c % 2, b).wait()

    return k(table)
